# Initial kernel scaffold; baseline (speedup 1.0000x reference)
import jax
import jax.numpy as jnp
from jax.experimental import pallas as pl


def kernel(x, scale, table):
    raise NotImplementedError("write your pallas kernel here")


# speedup vs baseline: 319.5336x; 319.5336x over previous
import functools

import jax
import jax.numpy as jnp
from jax import lax
from jax.experimental import pallas as pl
from jax.experimental.pallas import tpu as pltpu
from jax.experimental.pallas import tpu_sc as plsc

_NUM_EMB = 256
_IN_CH = 96
_MIN_SCALE = 2.5
_MAX_SCALE = 3.5

_B = 16
_HW = 56 * 56
_ROWS = _B * _IN_CH
_N = _ROWS * _HW

_LANES = 16
_NW = 32
_RPW = _ROWS // _NW
_RCH = 6
_NCHUNK = _RPW // _RCH


def _stats_body(x_ref, scale_ref, s_ref):
    xv = x_ref[...]
    s1 = jnp.sum(xv)
    s2 = jnp.sum(xv * xv)
    mx = jnp.max(jnp.abs(xv))
    n = jnp.float32(_N)
    var = (s2 - s1 * (s1 / n)) / (n - 1.0)
    std = jnp.sqrt(var)
    min_s = jnp.float32(_MIN_SCALE * 0.9) + (2.0 * std) * jnp.float32(1.0 - 0.9)
    max_s = jnp.float32(_MAX_SCALE * 0.9) + mx * jnp.float32(1.0 - 0.9)
    s_ref[...] = jnp.minimum(jnp.maximum(scale_ref[...], min_s), max_s)


_stats = pl.pallas_call(
    _stats_body,
    out_shape=jax.ShapeDtypeStruct((1, _IN_CH), jnp.float32),
)

def _remap_body(x_hbm, s_hbm, tab_hbm, out_hbm, tab_v, s_v, in_v, out_v,
                sem_in0, sem_in1, sem_out0, sem_out1):
    wid = lax.axis_index("s") * 2 + lax.axis_index("c")
    elt0 = wid * (_RPW * _HW)
    chunk_elts = _RCH * _HW
    pltpu.sync_copy(tab_hbm, tab_v)
    pltpu.sync_copy(s_hbm, s_v)
    sems_in = (sem_in0, sem_in1)
    sems_out = (sem_out0, sem_out1)

    def start_in(ch):
        buf = ch % 2
        return pltpu.async_copy(
            x_hbm.at[pl.ds(elt0 + ch * chunk_elts, chunk_elts)], in_v.at[buf],
            sems_in[buf])

    cps_in = {0: start_in(0)}
    cps_out = {}
    for ch in range(_NCHUNK):
        buf = ch % 2
        if ch + 1 < _NCHUNK:
            cps_in[ch + 1] = start_in(ch + 1)
        cps_in[ch].wait()
        if ch >= 2:
            cps_out[ch - 2].wait()
        for i in range(_RCH):
            r = wid * _RPW + ch * _RCH + i
            c = lax.rem(r, _IN_CH)
            cvec = jnp.broadcast_to(c, (_LANES,))
            sv = plsc.load_gather(s_v, [cvec])
            offv = jnp.broadcast_to(
                (c * _NUM_EMB).astype(jnp.float32), (_LANES,))

            @plsc.parallel_loop(0, _HW, _LANES, unroll=4)
            def _(o):
                xv = in_v[buf, pl.ds(i * _HW + o, _LANES)]
                xc = jnp.minimum(jnp.maximum(xv, -sv), sv)
                t0 = xc / sv
                t = ((t0 + 1.0) * 0.5) * 255.0 + offv
                li = t.astype(jnp.int32)
                lf = li.astype(jnp.float32)
                frac = t - lf
                ui = li + (t > lf).astype(jnp.int32)
                lv = plsc.load_gather(tab_v, [li])
                uv = plsc.load_gather(tab_v, [ui])
                out_v[buf, pl.ds(i * _HW + o, _LANES)] = (
                    frac * lv + (1.0 - frac) * uv)

        cps_out[ch] = pltpu.async_copy(
            out_v.at[buf], out_hbm.at[pl.ds(elt0 + ch * chunk_elts, chunk_elts)],
            sems_out[buf])
    cps_out[_NCHUNK - 2].wait()
    cps_out[_NCHUNK - 1].wait()


@functools.cache
def _build_remap():
    mesh = plsc.VectorSubcoreMesh(core_axis_name="c", subcore_axis_name="s")
    return pl.kernel(
        _remap_body,
        out_type=jax.ShapeDtypeStruct((_N,), jnp.float32),
        mesh=mesh,
        compiler_params=pltpu.CompilerParams(needs_layout_passes=False),
        scratch_types=[
            pltpu.VMEM((_NUM_EMB * _IN_CH,), jnp.float32),
            pltpu.VMEM((_IN_CH,), jnp.float32),
            pltpu.VMEM((2, _RCH * _HW), jnp.float32),
            pltpu.VMEM((2, _RCH * _HW), jnp.float32),
            pltpu.SemaphoreType.DMA,
            pltpu.SemaphoreType.DMA,
            pltpu.SemaphoreType.DMA,
            pltpu.SemaphoreType.DMA,
        ],
    )


def kernel(x, scale, table):
    s = _stats(x.reshape(1176, 4096), scale.reshape(1, _IN_CH))
    _remap = _build_remap()
    out = _remap(x.reshape(_N), s.reshape(_IN_CH), table.reshape(-1))
    return out.reshape(x.shape)


# speedup vs baseline: 343.8888x; 1.0762x over previous
import functools

import jax
import jax.numpy as jnp
from jax import lax
from jax.experimental import pallas as pl
from jax.experimental.pallas import tpu as pltpu
from jax.experimental.pallas import tpu_sc as plsc

_NUM_EMB = 256
_IN_CH = 96
_MIN_SCALE = 2.5
_MAX_SCALE = 3.5

_B = 16
_HW = 56 * 56
_ROWS = _B * _IN_CH
_N = _ROWS * _HW

_LANES = 16
_NW = 32
_CPW = _IN_CH // _NW
_BCH = 4
_NGRP = _B // _BCH
_NCHUNK = _CPW * _NGRP
_CHUNK_V = _BCH * _HW // _LANES


def _stats_body(x_ref, scale_ref, s_ref):
    xv = x_ref[...]
    s1 = jnp.sum(xv)
    s2 = jnp.sum(xv * xv)
    mx = jnp.max(jnp.abs(xv))
    n = jnp.float32(_N)
    var = (s2 - s1 * (s1 / n)) / (n - 1.0)
    std = jnp.sqrt(var)
    min_s = jnp.float32(_MIN_SCALE * 0.9) + (2.0 * std) * jnp.float32(1.0 - 0.9)
    max_s = jnp.float32(_MAX_SCALE * 0.9) + mx * jnp.float32(1.0 - 0.9)
    s_ref[...] = jnp.minimum(jnp.maximum(scale_ref[...], min_s), max_s)


_stats = pl.pallas_call(
    _stats_body,
    out_shape=jax.ShapeDtypeStruct((1, _IN_CH), jnp.float32),
)

def _remap_body(x_hbm, s_hbm, tab_hbm, out_hbm, tab_v, s_v, in_v0, in_v1,
                out_v0, out_v1, sem_in0, sem_in1, sem_out0, sem_out1):
    in_bufs = (in_v0, in_v1)
    out_bufs = (out_v0, out_v1)
    wid = lax.axis_index("s") * 2 + lax.axis_index("c")
    c0 = wid * _CPW
    pltpu.sync_copy(tab_hbm, tab_v)
    pltpu.sync_copy(s_hbm, s_v)
    sems_in = (sem_in0, sem_in1)
    sems_out = (sem_out0, sem_out1)

    def chunk_rows(ch):
        c = c0 + ch // _NGRP
        b0 = (ch % _NGRP) * _BCH
        return [(b0 + j) * _IN_CH + c for j in range(_BCH)]

    def start_in(ch):
        buf = ch % 2
        return [
            pltpu.async_copy(
                x_hbm.at[pl.ds(r * _HW, _HW)],
                in_bufs[buf].at[pl.ds(j * _HW, _HW)], sems_in[buf])
            for j, r in enumerate(chunk_rows(ch))
        ]

    def start_out(ch):
        buf = ch % 2
        return [
            pltpu.async_copy(
                out_bufs[buf].at[pl.ds(j * _HW, _HW)],
                out_hbm.at[pl.ds(r * _HW, _HW)], sems_out[buf])
            for j, r in enumerate(chunk_rows(ch))
        ]

    cps_in = {0: start_in(0)}
    cps_out = {}
    for ch in range(_NCHUNK):
        buf = ch % 2
        if ch + 1 < _NCHUNK:
            cps_in[ch + 1] = start_in(ch + 1)
        for cp in cps_in.pop(ch):
            cp.wait()
        if ch >= 2:
            for cp in cps_out.pop(ch - 2):
                cp.wait()
        if ch % _NGRP == 0:
            c = c0 + ch // _NGRP
            cvec = jnp.broadcast_to(c, (_LANES,))
            sv = plsc.load_gather(s_v, [cvec])
            offv = jnp.broadcast_to(
                (c * _NUM_EMB).astype(jnp.float32), (_LANES,))

        @plsc.parallel_loop(0, _BCH * _HW, _LANES, unroll=8)
        def _(o):
            xv = in_bufs[buf][pl.ds(o, _LANES)]
            xc = jnp.minimum(jnp.maximum(xv, -sv), sv)
            t0 = xc / sv
            t = ((t0 + 1.0) * 0.5) * 255.0 + offv
            li = t.astype(jnp.int32)
            lf = li.astype(jnp.float32)
            frac = t - lf
            ui = li + (t > lf).astype(jnp.int32)
            lv = plsc.load_gather(tab_v, [li])
            uv = plsc.load_gather(tab_v, [ui])
            out_bufs[buf][pl.ds(o, _LANES)] = uv + frac * (lv - uv)

        cps_out[ch] = start_out(ch)
    for ch in (_NCHUNK - 2, _NCHUNK - 1):
        for cp in cps_out.pop(ch):
            cp.wait()


@functools.cache
def _build_remap():
    mesh = plsc.VectorSubcoreMesh(core_axis_name="c", subcore_axis_name="s")
    return pl.kernel(
        _remap_body,
        out_type=jax.ShapeDtypeStruct((_N,), jnp.float32),
        mesh=mesh,
        compiler_params=pltpu.CompilerParams(needs_layout_passes=False),
        scratch_types=[
            pltpu.VMEM((_NUM_EMB * _IN_CH,), jnp.float32),
            pltpu.VMEM((_IN_CH,), jnp.float32),
            pltpu.VMEM((_BCH * _HW,), jnp.float32),
            pltpu.VMEM((_BCH * _HW,), jnp.float32),
            pltpu.VMEM((_BCH * _HW,), jnp.float32),
            pltpu.VMEM((_BCH * _HW,), jnp.float32),
            pltpu.SemaphoreType.DMA,
            pltpu.SemaphoreType.DMA,
            pltpu.SemaphoreType.DMA,
            pltpu.SemaphoreType.DMA,
        ],
    )


def kernel(x, scale, table):
    s = _stats(x.reshape(1176, 4096), scale.reshape(1, _IN_CH))
    _remap = _build_remap()
    out = _remap(x.reshape(_N), s.reshape(_IN_CH), table.reshape(-1))
    return out.reshape(x.shape)


# speedup vs baseline: 1133.8038x; 3.2970x over previous
import functools

import jax
import jax.numpy as jnp
import numpy as np
from jax import lax
from jax.experimental import pallas as pl
from jax.experimental.pallas import tpu as pltpu
from jax.experimental.pallas import tpu_sc as plsc

_NUM_EMB = 256
_IN_CH = 96
_MIN_SCALE = 2.5
_MAX_SCALE = 3.5

_B = 16
_HW = 56 * 56
_SP = _B * _HW
_N = _SP * _IN_CH

_LANES = 16
_NCHG = _IN_CH // _LANES
_NW = 32
_RPW = _SP // _NW
_RCH = 112
_NCHUNK = _RPW // _RCH
_NPAIR = _NCHUNK // 2


def _stats_body(x_ref, scale_ref, s_ref):
    xv = x_ref[...]
    s1 = jnp.sum(xv)
    s2 = jnp.sum(xv * xv)
    mx = jnp.max(jnp.abs(xv))
    n = jnp.float32(_N)
    var = (s2 - s1 * (s1 / n)) / (n - 1.0)
    std = jnp.sqrt(var)
    min_s = jnp.float32(_MIN_SCALE * 0.9) + (2.0 * std) * jnp.float32(1.0 - 0.9)
    max_s = jnp.float32(_MAX_SCALE * 0.9) + mx * jnp.float32(1.0 - 0.9)
    s_ref[...] = jnp.minimum(jnp.maximum(scale_ref[...], min_s), max_s)


_stats = pl.pallas_call(
    _stats_body,
    out_shape=jax.ShapeDtypeStruct((1, _IN_CH), jnp.float32),
    compiler_params=pltpu.CompilerParams(vmem_limit_bytes=60000 * 1024),
)


def _remap_body(x_hbm, s_hbm, tab_hbm, out_hbm, tab_v, s_v, in_v0, in_v1,
                out_v0, out_v1, sem_in0, sem_in1, sem_out0, sem_out1):
    in_bufs = (in_v0, in_v1)
    out_bufs = (out_v0, out_v1)
    sems_in = (sem_in0, sem_in1)
    sems_out = (sem_out0, sem_out1)
    wid = lax.axis_index("s") * 2 + lax.axis_index("c")
    row0 = wid * _RPW
    pltpu.sync_copy(tab_hbm, tab_v)
    pltpu.sync_copy(s_hbm, s_v)

    svs = [s_v[pl.ds(_LANES * j, _LANES)] for j in range(_NCHG)]
    lane = lax.iota(jnp.int32, _LANES).astype(jnp.float32)
    offvs = [
        (lane + jnp.float32(_LANES * j)) * jnp.float32(_NUM_EMB)
        for j in range(_NCHG)
    ]

    def start_in(ch, buf):
        start = pl.multiple_of(row0 + ch * _RCH, 8)
        pltpu.async_copy(
            x_hbm.at[pl.ds(start, _RCH)], in_bufs[buf], sems_in[buf])

    def start_out(ch, buf):
        start = pl.multiple_of(row0 + ch * _RCH, 8)
        pltpu.async_copy(
            out_bufs[buf], out_hbm.at[pl.ds(start, _RCH)], sems_out[buf])

    def wait_in(buf):
        pltpu.make_async_copy(
            x_hbm.at[pl.ds(0, _RCH)], in_bufs[buf], sems_in[buf]).wait()

    def wait_out(buf):
        pltpu.make_async_copy(
            out_bufs[buf], out_hbm.at[pl.ds(0, _RCH)], sems_out[buf]).wait()

    def compute(buf):
        in_v = in_bufs[buf]
        out_v = out_bufs[buf]

        @plsc.parallel_loop(0, _RCH, 1, unroll=2)
        def _(r):
            for j in range(_NCHG):
                sv = svs[j]
                xv = in_v[r, pl.ds(_LANES * j, _LANES)]
                xc = jnp.minimum(jnp.maximum(xv, -sv), sv)
                t0 = xc / sv
                t = ((t0 + 1.0) * 0.5) * 255.0 + offvs[j]
                li = t.astype(jnp.int32)
                lf = li.astype(jnp.float32)
                frac = t - lf
                ui = li + (t > lf).astype(jnp.int32)
                lv = plsc.load_gather(tab_v, [li])
                uv = plsc.load_gather(tab_v, [ui])
                out_v[r, pl.ds(_LANES * j, _LANES)] = uv + frac * (lv - uv)

    start_in(0, 0)

    def pair(k, carry):
        ch0 = 2 * k
        start_in(ch0 + 1, 1)
        wait_in(0)

        @pl.when(k > 0)
        def _():
            wait_out(0)

        compute(0)
        start_out(ch0, 0)

        @pl.when(k < _NPAIR - 1)
        def _():
            start_in(ch0 + 2, 0)

        wait_in(1)

        @pl.when(k > 0)
        def _():
            wait_out(1)

        compute(1)
        start_out(ch0 + 1, 1)
        return carry

    lax.fori_loop(0, _NPAIR, pair, None)
    wait_out(0)
    wait_out(1)


@functools.cache
def _build_remap():
    mesh = plsc.VectorSubcoreMesh(core_axis_name="c", subcore_axis_name="s")
    return pl.kernel(
        _remap_body,
        out_type=jax.ShapeDtypeStruct((_SP, _IN_CH), jnp.float32),
        mesh=mesh,
        compiler_params=pltpu.CompilerParams(needs_layout_passes=False),
        scratch_types=[
            pltpu.VMEM((_NUM_EMB * _IN_CH,), jnp.float32),
            pltpu.VMEM((_IN_CH,), jnp.float32),
            pltpu.VMEM((_RCH, _IN_CH), jnp.float32),
            pltpu.VMEM((_RCH, _IN_CH), jnp.float32),
            pltpu.VMEM((_RCH, _IN_CH), jnp.float32),
            pltpu.VMEM((_RCH, _IN_CH), jnp.float32),
            pltpu.SemaphoreType.DMA,
            pltpu.SemaphoreType.DMA,
            pltpu.SemaphoreType.DMA,
            pltpu.SemaphoreType.DMA,
        ],
    )


def kernel(x, scale, table):
    xt = x.transpose(0, 2, 3, 1).reshape(_SP, _IN_CH)
    s = _stats(xt, scale.reshape(1, _IN_CH))
    _remap = _build_remap()
    out = _remap(xt, s.reshape(_IN_CH), table.reshape(-1))
    return out.reshape(_B, 56, 56, _IN_CH).transpose(0, 3, 1, 2)
